# baseline (speedup 1.0000x reference)
import functools

import jax
import jax.numpy as jnp
from jax import lax
from jax.experimental import pallas as pl
from jax.experimental.pallas import tpu as pltpu
from jax.experimental.pallas import tpu_sc as plsc

B = 4096
C = 128
D = 128
BLK = 256
K = 10
KP = 16
EPS = 1e-08
ENTROPY_WEIGHT = 2.0
H = B // 2
NBLK2 = H // BLK
NWORK = 32
RPW = 128
KW = 5
LANES = 16


def _softmax(x):
    m = jnp.max(x, axis=-1, keepdims=True)
    e = jnp.exp(x - m)
    return e / jnp.sum(e, axis=-1, keepdims=True)


def _stage1_core(af_ref, gf_ref, anc_ref, nei_ref, aug_ref,
                 vec_out, g_out, scal_out, idxf_ref, wf_ref, augp_ref, ap_ref,
                 vec_ref, g_ref, gn_ref, s_ref):
    i = pl.program_id(0)

    @pl.when(i == 0)
    def _init():
        vec_ref[...] = jnp.zeros_like(vec_ref)
        g_ref[...] = jnp.zeros_like(g_ref)
        s_ref[0] = 0.0
        s_ref[1] = 0.0
        g = gf_ref[...]
        gn_ref[0:1, :] = jnp.sum(g * g, axis=1)[None, :]
        if aug_ref is not None:
            aug_prob = _softmax(aug_ref[...])
            augp_ref[...] = aug_prob
            vec_ref[3, :] = jnp.sum(aug_prob, axis=0)

    a_prob = _softmax(anc_ref[...])
    n_prob = _softmax(nei_ref[...])
    ap_ref[...] = a_prob

    sim = jnp.sum(a_prob * n_prob, axis=1)
    cons_part = jnp.sum(-jnp.maximum(jnp.log(sim), -100.0))
    vec_ref[0, :] += jnp.sum(a_prob, axis=0)
    vec_ref[1, :] += jnp.sum(a_prob * a_prob, axis=0)
    vec_ref[2, :] += jnp.sum(n_prob * n_prob, axis=0)
    g_ref[...] += jnp.dot(a_prob.T, n_prob, preferred_element_type=jnp.float32)

    a = af_ref[...]
    a_nrm = jnp.sum(a * a, axis=1, keepdims=True)
    d2 = a_nrm + gn_ref[0:1, :] - 2.0 * jnp.dot(
        a, gf_ref[...].T, preferred_element_type=jnp.float32)
    d2c = jnp.maximum(d2, 0.0)

    col = jax.lax.broadcasted_iota(jnp.int32, (BLK, B), 1)
    bits = jax.lax.bitcast_convert_type(d2c, jnp.int32)
    packed = jax.lax.bitcast_convert_type(
        jnp.bitwise_or(jnp.bitwise_and(bits, jnp.int32(~0xFFF)), col),
        jnp.float32)
    fmax = jnp.float32(3.4028235e38)
    mins = []
    for k in range(K):
        m = jnp.min(packed, axis=1, keepdims=True)
        packed = jnp.where(packed == m, fmax, packed)
        mins.append(m)

    vals_p = jnp.concatenate(mins, axis=1)
    bits_k = jax.lax.bitcast_convert_type(vals_p, jnp.int32)
    v2 = jax.lax.bitcast_convert_type(
        jnp.bitwise_and(bits_k, jnp.int32(~0xFFF)), jnp.float32)
    r = jnp.sqrt(v2[:, 1:2])
    w_tiny = jnp.clip(2.0 - jnp.sqrt(v2) / r, 0.0, 1.0)
    s_ref[0] += cons_part
    s_ref[1] += jnp.sum(w_tiny)

    idx_f = jnp.bitwise_and(bits_k, jnp.int32(0xFFF)).astype(jnp.float32)
    pad3 = jnp.zeros((BLK, 3), jnp.float32)
    idx16 = jnp.concatenate(
        [idx_f[:, :KW], pad3, idx_f[:, KW:], pad3], axis=1)
    w16 = jnp.concatenate(
        [w_tiny[:, :KW], pad3, w_tiny[:, KW:], pad3], axis=1)
    rid = jax.lax.broadcasted_iota(jnp.int32, (BLK, BLK), 0)
    cid = jax.lax.broadcasted_iota(jnp.int32, (BLK, BLK), 1)
    eye = jnp.where(rid == cid, 1.0, 0.0).astype(jnp.float32)
    idxf_ref[...] = jnp.dot(
        idx16.T, eye, preferred_element_type=jnp.float32)
    wf_ref[...] = jnp.dot(w16.T, eye, preferred_element_type=jnp.float32)

    @pl.when(i == NBLK2 - 1)
    def _fin():
        vec_out[...] = vec_ref[...]
        g_out[...] = g_ref[...]
        lane = jax.lax.broadcasted_iota(jnp.int32, (1, C), 1)
        scal_out[...] = jnp.where(
            lane == 0, s_ref[0], jnp.where(lane == 1, s_ref[1], 0.0))


def _stage1_a(af_ref, gf_ref, anc_ref, nei_ref, aug_ref,
              vec_out, g_out, scal_out, idxf_ref, wf_ref, augp_ref, ap_ref,
              vec_ref, g_ref, gn_ref, s_ref):
    _stage1_core(af_ref, gf_ref, anc_ref, nei_ref, aug_ref,
                 vec_out, g_out, scal_out, idxf_ref, wf_ref, augp_ref, ap_ref,
                 vec_ref, g_ref, gn_ref, s_ref)


def _stage1_b(af_ref, gf_ref, anc_ref, nei_ref,
              vec_out, g_out, scal_out, idxf_ref, wf_ref, ap_ref,
              vec_ref, g_ref, gn_ref, s_ref):
    _stage1_core(af_ref, gf_ref, anc_ref, nei_ref, None,
                 vec_out, g_out, scal_out, idxf_ref, wf_ref, None, ap_ref,
                 vec_ref, g_ref, gn_ref, s_ref)


_SC_MESH = plsc.VectorSubcoreMesh(core_axis_name="c", subcore_axis_name="s")


@functools.partial(
    pl.kernel,
    mesh=_SC_MESH,
    out_type=jax.ShapeDtypeStruct((NWORK, C), jnp.float32),
    scratch_types=[
        pltpu.VMEM((8, RPW), jnp.float32),
        pltpu.VMEM((8, RPW), jnp.int32),
        pltpu.VMEM((8, RPW), jnp.float32),
        pltpu.VMEM((KW * RPW, C), jnp.float32),
        pltpu.VMEM((RPW, C), jnp.float32),
        pltpu.VMEM((C,), jnp.float32),
        pltpu.SemaphoreType.DMA,
    ],
)
def _sc_gather(idxf_hbm, wf_hbm, augp_hbm, ap_hbm, out_hbm,
               idxf_v, idx_v, w_v, rows_v, a_v, acc_v, sem):
    s_ax = lax.axis_index("s")
    c_ax = lax.axis_index("c")
    wid = s_ax * 2 + c_ax
    base = s_ax * RPW
    koff = c_ax * 8

    pltpu.sync_copy(idxf_hbm.at[pl.ds(koff, 8), pl.ds(base, RPW)], idxf_v)
    pltpu.sync_copy(wf_hbm.at[pl.ds(koff, 8), pl.ds(base, RPW)], w_v)
    pltpu.sync_copy(ap_hbm.at[pl.ds(base, RPW)], a_v)
    for k in range(KW):
        for c in range(RPW // LANES):
            sl = pl.ds(c * LANES, LANES)
            idx_v[k, sl] = idxf_v[k, sl].astype(jnp.int32)

    copies = [
        pltpu.async_copy(augp_hbm.at[idx_v.at[k]],
                         rows_v.at[pl.ds(k * RPW, RPW)], sem)
        for k in range(KW)
    ]
    for cp in copies:
        cp.wait()

    acc0 = (jnp.zeros((LANES,), jnp.float32),) * (C // LANES)

    def k_body(k, acc):
        def q_body(q, acc_in):
            wvec = w_v[k, pl.ds(q * LANES, LANES)]
            accs = list(acc_in)
            for l in range(LANES):
                r = q * LANES + l
                g = k * RPW + r
                wl = wvec[l]
                for c in range(C // LANES):
                    sl = pl.ds(c * LANES, LANES)
                    accs[c] = accs[c] + wl * (rows_v[g, sl] * a_v[r, sl])
            return tuple(accs)

        return lax.fori_loop(0, RPW // LANES, q_body, acc)

    acc = lax.fori_loop(0, KW, k_body, acc0)
    for c in range(C // LANES):
        acc_v[pl.ds(c * LANES, LANES)] = acc[c]
    pltpu.sync_copy(acc_v, out_hbm.at[wid])


def _fin_kernel(veca_ref, vecb_ref, ga_ref, gb_ref, sa_ref, sb_ref,
                wipa_ref, wipb_ref, total_ref, cons_ref, ent_ref):
    bsq = jnp.float32(B) * jnp.float32(B)
    vec = veca_ref[...] + vecb_ref[...]
    sc = sa_ref[...] + sb_ref[...]
    cons_sum = jnp.sum(sc[0:1, 0:1])
    sum_w = jnp.sum(sc[0:1, 1:2])
    wip = jnp.sum(wipa_ref[...]) + jnp.sum(wipb_ref[...])

    colsum_a = vec[0, :]
    mean_ip = jnp.sum(colsum_a * vec[3, :]) / bsq
    glob = 1.0 - mean_ip - sum_w / bsq + 2.0 * wip / bsq

    mprob = jnp.maximum(colsum_a / jnp.float32(B), EPS)
    ent = -jnp.sum(mprob * jnp.log(mprob))

    na = jnp.maximum(jnp.sqrt(vec[1, :]), 1e-12)
    np_ = jnp.maximum(jnp.sqrt(vec[2, :]), 1e-12)
    sim_cc = (ga_ref[...] + gb_ref[...]) / (na[:, None] * np_[None, :])
    mx = jnp.max(sim_cc, axis=1, keepdims=True)
    lse = jnp.log(jnp.sum(jnp.exp(sim_cc - mx), axis=1, keepdims=True)) + mx
    rid = jax.lax.broadcasted_iota(jnp.int32, (C, C), 0)
    cid = jax.lax.broadcasted_iota(jnp.int32, (C, C), 1)
    diag_sum = jnp.sum(jnp.where(rid == cid, sim_cc, 0.0))
    ce = (jnp.sum(lse) - diag_sum) / jnp.float32(C)

    cons = cons_sum / jnp.float32(B)
    total_ref[...] = jnp.reshape(
        cons - ENTROPY_WEIGHT * ent + ce + glob, (1, 1))
    cons_ref[...] = jnp.reshape(cons, (1, 1))
    ent_ref[...] = jnp.reshape(ent, (1, 1))


@jax.jit
def kernel(anchors_features, augments_features, anchors, neighbors, augments):
    fullD = pl.BlockSpec((B, D), lambda i: (0, 0))
    fullC = pl.BlockSpec((B, C), lambda i: (0, 0))
    out8c = pl.BlockSpec((8, C), lambda i: (0, 0))
    outcc = pl.BlockSpec((C, C), lambda i: (0, 0))
    out1c = pl.BlockSpec((1, C), lambda i: (0, 0))
    kblk = pl.BlockSpec((KP, BLK), lambda i: (0, i))

    def half_specs(off):
        return [
            pl.BlockSpec((BLK, D), lambda i: (i + off, 0)), fullD,
            pl.BlockSpec((BLK, C), lambda i: (i + off, 0)),
            pl.BlockSpec((BLK, C), lambda i: (i + off, 0)),
        ]

    common_shapes = [
        jax.ShapeDtypeStruct((8, C), jnp.float32),
        jax.ShapeDtypeStruct((C, C), jnp.float32),
        jax.ShapeDtypeStruct((1, C), jnp.float32),
        jax.ShapeDtypeStruct((KP, H), jnp.float32),
        jax.ShapeDtypeStruct((KP, H), jnp.float32),
    ]
    common_outspecs = [out8c, outcc, out1c, kblk, kblk]
    blkC = pl.BlockSpec((BLK, C), lambda i: (i, 0))
    scratches = [
        pltpu.VMEM((8, C), jnp.float32),
        pltpu.VMEM((C, C), jnp.float32),
        pltpu.VMEM((8, B), jnp.float32),
        pltpu.SMEM((4,), jnp.float32),
    ]

    veca, ga, sa, idxfa, wfa, augp, apa = pl.pallas_call(
        _stage1_a,
        grid=(NBLK2,),
        in_specs=half_specs(0) + [fullC],
        out_specs=common_outspecs + [fullC, blkC],
        out_shape=common_shapes + [
            jax.ShapeDtypeStruct((B, C), jnp.float32),
            jax.ShapeDtypeStruct((H, C), jnp.float32),
        ],
        scratch_shapes=scratches,
    )(anchors_features, augments_features, anchors, neighbors, augments)

    wipa = _sc_gather(idxfa, wfa, augp, apa)

    vecb, gb, sb, idxfb, wfb, apb = pl.pallas_call(
        _stage1_b,
        grid=(NBLK2,),
        in_specs=half_specs(NBLK2),
        out_specs=common_outspecs + [blkC],
        out_shape=common_shapes + [jax.ShapeDtypeStruct((H, C), jnp.float32)],
        scratch_shapes=scratches,
    )(anchors_features, augments_features, anchors, neighbors)

    wipb = _sc_gather(idxfb, wfb, augp, apb)

    total, cons, ent = pl.pallas_call(
        _fin_kernel,
        out_shape=[jax.ShapeDtypeStruct((1, 1), jnp.float32)] * 3,
    )(veca, vecb, ga, gb, sa, sb, wipa, wipb)
    return (total[0, 0], cons[0, 0], ent[0, 0])


# speedup vs baseline: 1.1337x; 1.1337x over previous
import functools

import jax
import jax.numpy as jnp
from jax import lax
from jax.experimental import pallas as pl
from jax.experimental.pallas import tpu as pltpu
from jax.experimental.pallas import tpu_sc as plsc

B = 4096
C = 128
D = 128
BLK = 256
K = 10
KP = 16
EPS = 1e-08
ENTROPY_WEIGHT = 2.0
NBLK = B // BLK
NWORK = 32
RPW = B // NWORK
LANES = 16


def _softmax(x):
    m = jnp.max(x, axis=-1, keepdims=True)
    e = jnp.exp(x - m)
    return e / jnp.sum(e, axis=-1, keepdims=True)


def _main_kernel(af_ref, gf_ref, anc_ref, nei_ref, aug_ref,
                 tp_ref, cons_ref, ent_ref, idxf_ref, wf_ref, augp_ref, ap_ref,
                 vec_ref, g_ref, aug_scr, gn_ref, s_ref):
    i = pl.program_id(0)

    @pl.when(i == 0)
    def _init():
        vec_ref[...] = jnp.zeros_like(vec_ref)
        g_ref[...] = jnp.zeros_like(g_ref)
        s_ref[0] = 0.0
        s_ref[1] = 0.0
        aug_prob = _softmax(aug_ref[...])
        aug_scr[...] = aug_prob
        vec_ref[3, :] = jnp.sum(aug_prob, axis=0)
        g = gf_ref[...]
        gn_ref[0:1, :] = jnp.sum(g * g, axis=1)[None, :]

    a_prob = _softmax(anc_ref[...])
    n_prob = _softmax(nei_ref[...])
    ap_ref[...] = a_prob
    augp_ref[...] = aug_scr[pl.ds(i * BLK, BLK), :]

    sim = jnp.sum(a_prob * n_prob, axis=1)
    cons_part = jnp.sum(-jnp.maximum(jnp.log(sim), -100.0))
    vec_ref[0, :] += jnp.sum(a_prob, axis=0)
    vec_ref[1, :] += jnp.sum(a_prob * a_prob, axis=0)
    vec_ref[2, :] += jnp.sum(n_prob * n_prob, axis=0)
    g_ref[...] += jnp.dot(a_prob.T, n_prob, preferred_element_type=jnp.float32)

    a = af_ref[...]
    a_nrm = jnp.sum(a * a, axis=1, keepdims=True)
    d2 = a_nrm + gn_ref[0:1, :] - 2.0 * jnp.dot(
        a, gf_ref[...].T, preferred_element_type=jnp.float32)
    d2c = jnp.maximum(d2, 0.0)

    col = jax.lax.broadcasted_iota(jnp.int32, (BLK, B), 1)
    bits = jax.lax.bitcast_convert_type(d2c, jnp.int32)
    packed = jax.lax.bitcast_convert_type(
        jnp.bitwise_or(jnp.bitwise_and(bits, jnp.int32(~0xFFF)), col),
        jnp.float32)
    fmax = jnp.float32(3.4028235e38)
    mins = []
    for k in range(K):
        m = jnp.min(packed, axis=1, keepdims=True)
        packed = jnp.where(packed == m, fmax, packed)
        mins.append(m)

    vals_p = jnp.concatenate(mins, axis=1)
    bits_k = jax.lax.bitcast_convert_type(vals_p, jnp.int32)
    v2 = jax.lax.bitcast_convert_type(
        jnp.bitwise_and(bits_k, jnp.int32(~0xFFF)), jnp.float32)
    r = jnp.sqrt(v2[:, 1:2])
    w_tiny = jnp.clip(2.0 - jnp.sqrt(v2) / r, 0.0, 1.0)
    s_ref[0] += cons_part
    s_ref[1] += jnp.sum(w_tiny)

    idx_f = jnp.bitwise_and(bits_k, jnp.int32(0xFFF)).astype(jnp.float32)
    pad = jnp.zeros((BLK, KP - K), jnp.float32)
    idx16 = jnp.concatenate([idx_f, pad], axis=1)
    w16 = jnp.concatenate([w_tiny, pad], axis=1)
    rid = jax.lax.broadcasted_iota(jnp.int32, (BLK, BLK), 0)
    cid = jax.lax.broadcasted_iota(jnp.int32, (BLK, BLK), 1)
    eye = jnp.where(rid == cid, 1.0, 0.0).astype(jnp.float32)
    idxf_ref[...] = jnp.dot(idx16.T, eye, preferred_element_type=jnp.float32)
    wf_ref[...] = jnp.dot(w16.T, eye, preferred_element_type=jnp.float32)

    @pl.when(i == NBLK - 1)
    def _fin():
        bsq = jnp.float32(B) * jnp.float32(B)
        colsum_a = vec_ref[0, :]
        mean_ip = jnp.sum(colsum_a * vec_ref[3, :]) / bsq
        glob_part = 1.0 - mean_ip - s_ref[1] / bsq

        mprob = jnp.maximum(colsum_a / jnp.float32(B), EPS)
        ent = -jnp.sum(mprob * jnp.log(mprob))

        na = jnp.maximum(jnp.sqrt(vec_ref[1, :]), 1e-12)
        np_ = jnp.maximum(jnp.sqrt(vec_ref[2, :]), 1e-12)
        sim_cc = g_ref[...] / (na[:, None] * np_[None, :])
        mx = jnp.max(sim_cc, axis=1, keepdims=True)
        lse = jnp.log(jnp.sum(jnp.exp(sim_cc - mx), axis=1, keepdims=True)) + mx
        rid2 = jax.lax.broadcasted_iota(jnp.int32, (C, C), 0)
        cid2 = jax.lax.broadcasted_iota(jnp.int32, (C, C), 1)
        diag_sum = jnp.sum(jnp.where(rid2 == cid2, sim_cc, 0.0))
        ce = (jnp.sum(lse) - diag_sum) / jnp.float32(C)

        cons = s_ref[0] / jnp.float32(B)
        tp_ref[...] = jnp.reshape(
            cons - ENTROPY_WEIGHT * ent + ce + glob_part, (1, 1))
        cons_ref[...] = jnp.reshape(cons, (1, 1))
        ent_ref[...] = jnp.reshape(ent, (1, 1))


_SC_MESH = plsc.VectorSubcoreMesh(core_axis_name="c", subcore_axis_name="s")


@functools.partial(
    pl.kernel,
    mesh=_SC_MESH,
    out_type=jax.ShapeDtypeStruct((NWORK, C), jnp.float32),
    scratch_types=[
        pltpu.VMEM((RPW,), jnp.float32),
        pltpu.VMEM((RPW,), jnp.int32),
        pltpu.VMEM((RPW,), jnp.float32),
        pltpu.VMEM((RPW, C), jnp.float32),
        pltpu.VMEM((RPW, C), jnp.float32),
        pltpu.VMEM((C,), jnp.float32),
        pltpu.SemaphoreType.DMA,
    ],
)
def _sc_gather(idxf_hbm, wf_hbm, augp_hbm, ap_hbm, out_hbm,
               idxf_v, idx_v, w_v, rows_v, a_v, acc_v, sem):
    wid = lax.axis_index("s") * 2 + lax.axis_index("c")
    base = wid * RPW
    pltpu.sync_copy(ap_hbm.at[pl.ds(base, RPW)], a_v)

    acc0 = (jnp.zeros((LANES,), jnp.float32),) * (C // LANES)

    def k_body(k, acc):
        pltpu.sync_copy(idxf_hbm.at[k, pl.ds(base, RPW)], idxf_v)
        pltpu.sync_copy(wf_hbm.at[k, pl.ds(base, RPW)], w_v)
        for c in range(RPW // LANES):
            sl = pl.ds(c * LANES, LANES)
            idx_v[sl] = idxf_v[sl].astype(jnp.int32)
        pltpu.async_copy(augp_hbm.at[idx_v], rows_v, sem).wait()

        def q_body(q, acc_in):
            wvec = w_v[pl.ds(q * LANES, LANES)]
            accs = list(acc_in)
            for l in range(LANES):
                r = q * LANES + l
                wl = wvec[l]
                for c in range(C // LANES):
                    sl = pl.ds(c * LANES, LANES)
                    accs[c] = accs[c] + wl * (rows_v[r, sl] * a_v[r, sl])
            return tuple(accs)

        return lax.fori_loop(0, RPW // LANES, q_body, acc)

    acc = lax.fori_loop(0, K, k_body, acc0)
    for c in range(C // LANES):
        acc_v[pl.ds(c * LANES, LANES)] = acc[c]
    pltpu.sync_copy(acc_v, out_hbm.at[wid])


def _fin_kernel(tp_ref, wip_ref, out_ref):
    bsq = jnp.float32(B) * jnp.float32(B)
    out_ref[...] = tp_ref[...] + 2.0 * jnp.sum(wip_ref[...]) / bsq


@jax.jit
def kernel(anchors_features, augments_features, anchors, neighbors, augments):
    full = pl.BlockSpec((B, D), lambda i: (0, 0))
    blk = pl.BlockSpec((BLK, C), lambda i: (i, 0))
    out1 = pl.BlockSpec((1, 1), lambda i: (0, 0))
    kblk = pl.BlockSpec((KP, BLK), lambda i: (0, i))
    tp, cons, ent, idxf, wf, augp, ap = pl.pallas_call(
        _main_kernel,
        grid=(NBLK,),
        in_specs=[pl.BlockSpec((BLK, D), lambda i: (i, 0)), full, blk, blk, full],
        out_specs=[out1, out1, out1, kblk, kblk, blk, blk],
        out_shape=[
            jax.ShapeDtypeStruct((1, 1), jnp.float32),
            jax.ShapeDtypeStruct((1, 1), jnp.float32),
            jax.ShapeDtypeStruct((1, 1), jnp.float32),
            jax.ShapeDtypeStruct((KP, B), jnp.float32),
            jax.ShapeDtypeStruct((KP, B), jnp.float32),
            jax.ShapeDtypeStruct((B, C), jnp.float32),
            jax.ShapeDtypeStruct((B, C), jnp.float32),
        ],
        scratch_shapes=[
            pltpu.VMEM((8, C), jnp.float32),
            pltpu.VMEM((C, C), jnp.float32),
            pltpu.VMEM((B, C), jnp.float32),
            pltpu.VMEM((8, B), jnp.float32),
            pltpu.SMEM((4,), jnp.float32),
        ],
    )(anchors_features, augments_features, anchors, neighbors, augments)

    wip_part = _sc_gather(idxf, wf, augp, ap)

    total = pl.pallas_call(
        _fin_kernel,
        out_shape=jax.ShapeDtypeStruct((1, 1), jnp.float32),
    )(tp, wip_part)
    return (total[0, 0], cons[0, 0], ent[0, 0])


# speedup vs baseline: 1.2332x; 1.0877x over previous
import functools

import jax
import jax.numpy as jnp
from jax import lax
from jax.experimental import pallas as pl
from jax.experimental.pallas import tpu as pltpu
from jax.experimental.pallas import tpu_sc as plsc

B = 4096
C = 128
D = 128
BLK = 256
K = 10
KP = 16
EPS = 1e-08
ENTROPY_WEIGHT = 2.0
NBLK = B // BLK
NWORK = 32
RPW = B // NWORK
LANES = 16


def _softmax(x):
    m = jnp.max(x, axis=-1, keepdims=True)
    e = jnp.exp(x - m)
    return e / jnp.sum(e, axis=-1, keepdims=True)


def _main_kernel(af_ref, gf_ref, anc_ref, nei_ref, aug_ref,
                 tp_ref, cons_ref, ent_ref, idxf_ref, wf_ref, augp_ref, ap_ref,
                 vec_ref, g_ref, aug_scr, gn_ref, s_ref):
    i = pl.program_id(0)

    @pl.when(i == 0)
    def _init():
        vec_ref[...] = jnp.zeros_like(vec_ref)
        g_ref[...] = jnp.zeros_like(g_ref)
        s_ref[0] = 0.0
        s_ref[1] = 0.0
        aug_prob = _softmax(aug_ref[...])
        aug_scr[...] = aug_prob
        vec_ref[3, :] = jnp.sum(aug_prob, axis=0)
        g = gf_ref[...]
        gn_ref[0:1, :] = jnp.sum(g * g, axis=1)[None, :]

    a_prob = _softmax(anc_ref[...])
    n_prob = _softmax(nei_ref[...])
    ap_ref[...] = a_prob
    augp_ref[...] = aug_scr[pl.ds(i * BLK, BLK), :]

    sim = jnp.sum(a_prob * n_prob, axis=1)
    cons_part = jnp.sum(-jnp.maximum(jnp.log(sim), -100.0))
    vec_ref[0, :] += jnp.sum(a_prob, axis=0)
    vec_ref[1, :] += jnp.sum(a_prob * a_prob, axis=0)
    vec_ref[2, :] += jnp.sum(n_prob * n_prob, axis=0)
    g_ref[...] += jnp.dot(a_prob.T, n_prob, preferred_element_type=jnp.float32)

    a = af_ref[...]
    a_nrm = jnp.sum(a * a, axis=1, keepdims=True)
    d2 = a_nrm + gn_ref[0:1, :] - 2.0 * jnp.dot(
        a, gf_ref[...].T, preferred_element_type=jnp.float32)
    d2c = jnp.maximum(d2, 0.0)

    col = jax.lax.broadcasted_iota(jnp.int32, (BLK, B), 1)
    bits = jax.lax.bitcast_convert_type(d2c, jnp.int32)
    packed = jax.lax.bitcast_convert_type(
        jnp.bitwise_or(jnp.bitwise_and(bits, jnp.int32(~0xFFF)), col),
        jnp.float32)
    fmax = jnp.float32(3.4028235e38)
    mins = []
    for k in range(K):
        m = jnp.min(packed, axis=1, keepdims=True)
        packed = jnp.where(packed == m, fmax, packed)
        mins.append(m)

    vals_p = jnp.concatenate(mins, axis=1)
    bits_k = jax.lax.bitcast_convert_type(vals_p, jnp.int32)
    v2 = jax.lax.bitcast_convert_type(
        jnp.bitwise_and(bits_k, jnp.int32(~0xFFF)), jnp.float32)
    r = jnp.sqrt(v2[:, 1:2])
    w_tiny = jnp.clip(2.0 - jnp.sqrt(v2) / r, 0.0, 1.0)
    s_ref[0] += cons_part
    s_ref[1] += jnp.sum(w_tiny)

    idx_f = jnp.bitwise_and(bits_k, jnp.int32(0xFFF)).astype(jnp.float32)
    pad = jnp.zeros((BLK, KP - K), jnp.float32)
    idx16 = jnp.concatenate([idx_f, pad], axis=1)
    w16 = jnp.concatenate([w_tiny, pad], axis=1)
    rid = jax.lax.broadcasted_iota(jnp.int32, (BLK, BLK), 0)
    cid = jax.lax.broadcasted_iota(jnp.int32, (BLK, BLK), 1)
    eye = jnp.where(rid == cid, 1.0, 0.0).astype(jnp.float32)
    idxf_ref[...] = jnp.dot(idx16.T, eye, preferred_element_type=jnp.float32)
    wf_ref[...] = jnp.dot(w16.T, eye, preferred_element_type=jnp.float32)

    @pl.when(i == NBLK - 1)
    def _fin():
        bsq = jnp.float32(B) * jnp.float32(B)
        colsum_a = vec_ref[0, :]
        mean_ip = jnp.sum(colsum_a * vec_ref[3, :]) / bsq
        glob_part = 1.0 - mean_ip - s_ref[1] / bsq

        mprob = jnp.maximum(colsum_a / jnp.float32(B), EPS)
        ent = -jnp.sum(mprob * jnp.log(mprob))

        na = jnp.maximum(jnp.sqrt(vec_ref[1, :]), 1e-12)
        np_ = jnp.maximum(jnp.sqrt(vec_ref[2, :]), 1e-12)
        sim_cc = g_ref[...] / (na[:, None] * np_[None, :])
        mx = jnp.max(sim_cc, axis=1, keepdims=True)
        lse = jnp.log(jnp.sum(jnp.exp(sim_cc - mx), axis=1, keepdims=True)) + mx
        rid2 = jax.lax.broadcasted_iota(jnp.int32, (C, C), 0)
        cid2 = jax.lax.broadcasted_iota(jnp.int32, (C, C), 1)
        diag_sum = jnp.sum(jnp.where(rid2 == cid2, sim_cc, 0.0))
        ce = (jnp.sum(lse) - diag_sum) / jnp.float32(C)

        cons = s_ref[0] / jnp.float32(B)
        tp_ref[...] = jnp.reshape(
            cons - ENTROPY_WEIGHT * ent + ce + glob_part, (1, 1))
        cons_ref[...] = jnp.reshape(cons, (1, 1))
        ent_ref[...] = jnp.reshape(ent, (1, 1))


_SC_MESH = plsc.VectorSubcoreMesh(core_axis_name="c", subcore_axis_name="s")


@functools.partial(
    pl.kernel,
    mesh=_SC_MESH,
    out_type=jax.ShapeDtypeStruct((NWORK, C), jnp.float32),
    scratch_types=[
        pltpu.VMEM((KP, RPW), jnp.float32),
        pltpu.VMEM((KP, RPW), jnp.int32),
        pltpu.VMEM((KP, RPW), jnp.float32),
        pltpu.VMEM((2 * RPW, C), jnp.float32),
        pltpu.VMEM((RPW, C), jnp.float32),
        pltpu.VMEM((C,), jnp.float32),
        pltpu.SemaphoreType.DMA,
        pltpu.SemaphoreType.DMA,
    ],
)
def _sc_gather(idxf_hbm, wf_hbm, augp_hbm, ap_hbm, out_hbm,
               idxf_v, idx_v, w_v, rows_v, a_v, acc_v, sem0, sem1):
    wid = lax.axis_index("s") * 2 + lax.axis_index("c")
    base = wid * RPW

    pltpu.sync_copy(idxf_hbm.at[pl.ds(0, KP), pl.ds(base, RPW)], idxf_v)
    pltpu.sync_copy(wf_hbm.at[pl.ds(0, KP), pl.ds(base, RPW)], w_v)
    pltpu.sync_copy(ap_hbm.at[pl.ds(base, RPW)], a_v)
    for k in range(K):
        for c in range(RPW // LANES):
            sl = pl.ds(c * LANES, LANES)
            idx_v[k, sl] = idxf_v[k, sl].astype(jnp.int32)

    sems = (sem0, sem1)

    def issue(k):
        buf = k % 2
        return pltpu.async_copy(augp_hbm.at[idx_v.at[k]],
                                rows_v.at[pl.ds(buf * RPW, RPW)], sems[buf])

    acc = (jnp.zeros((LANES,), jnp.float32),) * (C // LANES)
    pending = {0: issue(0)}
    for k in range(K):
        if k + 1 < K:
            pending[(k + 1) % 2] = issue(k + 1)
        pending[k % 2].wait()
        buf = k % 2

        def q_body(q, acc_in, _k=k, _buf=buf):
            qsl = pl.ds(q * LANES, LANES)
            wvec = w_v[_k, qsl]
            accs = list(acc_in)
            for l in range(LANES):
                r = q * LANES + l
                wl = wvec[l]
                for c in range(C // LANES):
                    sl = pl.ds(c * LANES, LANES)
                    accs[c] = accs[c] + wl * (
                        rows_v[_buf * RPW + r, sl] * a_v[r, sl])
            return tuple(accs)

        acc = lax.fori_loop(0, RPW // LANES, q_body, acc)
    for c in range(C // LANES):
        acc_v[pl.ds(c * LANES, LANES)] = acc[c]
    pltpu.sync_copy(acc_v, out_hbm.at[wid])


def _fin_kernel(tp_ref, wip_ref, out_ref):
    bsq = jnp.float32(B) * jnp.float32(B)
    out_ref[...] = tp_ref[...] + 2.0 * jnp.sum(wip_ref[...]) / bsq


@jax.jit
def kernel(anchors_features, augments_features, anchors, neighbors, augments):
    full = pl.BlockSpec((B, D), lambda i: (0, 0))
    blk = pl.BlockSpec((BLK, C), lambda i: (i, 0))
    out1 = pl.BlockSpec((1, 1), lambda i: (0, 0))
    kblk = pl.BlockSpec((KP, BLK), lambda i: (0, i))
    tp, cons, ent, idxf, wf, augp, ap = pl.pallas_call(
        _main_kernel,
        grid=(NBLK,),
        in_specs=[pl.BlockSpec((BLK, D), lambda i: (i, 0)), full, blk, blk, full],
        out_specs=[out1, out1, out1, kblk, kblk, blk, blk],
        out_shape=[
            jax.ShapeDtypeStruct((1, 1), jnp.float32),
            jax.ShapeDtypeStruct((1, 1), jnp.float32),
            jax.ShapeDtypeStruct((1, 1), jnp.float32),
            jax.ShapeDtypeStruct((KP, B), jnp.float32),
            jax.ShapeDtypeStruct((KP, B), jnp.float32),
            jax.ShapeDtypeStruct((B, C), jnp.float32),
            jax.ShapeDtypeStruct((B, C), jnp.float32),
        ],
        scratch_shapes=[
            pltpu.VMEM((8, C), jnp.float32),
            pltpu.VMEM((C, C), jnp.float32),
            pltpu.VMEM((B, C), jnp.float32),
            pltpu.VMEM((8, B), jnp.float32),
            pltpu.SMEM((4,), jnp.float32),
        ],
    )(anchors_features, augments_features, anchors, neighbors, augments)

    wip_part = _sc_gather(idxf, wf, augp, ap)

    total = pl.pallas_call(
        _fin_kernel,
        out_shape=jax.ShapeDtypeStruct((1, 1), jnp.float32),
    )(tp, wip_part)
    return (total[0, 0], cons[0, 0], ent[0, 0])


# speedup vs baseline: 1.2608x; 1.0224x over previous
import functools

import jax
import jax.numpy as jnp
from jax import lax
from jax.experimental import pallas as pl
from jax.experimental.pallas import tpu as pltpu
from jax.experimental.pallas import tpu_sc as plsc

B = 4096
C = 128
D = 128
BLK = 512
K = 10
KP = 16
EPS = 1e-08
ENTROPY_WEIGHT = 2.0
NBLK = B // BLK
NWORK = 32
RPW = B // NWORK
LANES = 16


def _softmax(x):
    m = jnp.max(x, axis=-1, keepdims=True)
    e = jnp.exp(x - m)
    return e / jnp.sum(e, axis=-1, keepdims=True)


def _main_kernel(af_ref, gf_ref, anc_ref, nei_ref, aug_ref,
                 tp_ref, cons_ref, ent_ref, idxf_ref, wf_ref, augp_ref, ap_ref,
                 vec_ref, g_ref, aug_scr, gn_ref, s_ref):
    i = pl.program_id(0)

    @pl.when(i == 0)
    def _init():
        vec_ref[...] = jnp.zeros_like(vec_ref)
        g_ref[...] = jnp.zeros_like(g_ref)
        s_ref[0] = 0.0
        s_ref[1] = 0.0
        aug_prob = _softmax(aug_ref[...])
        aug_scr[...] = aug_prob
        vec_ref[3, :] = jnp.sum(aug_prob, axis=0)
        g = gf_ref[...]
        gn_ref[0:1, :] = jnp.sum(g * g, axis=1)[None, :]

    a_prob = _softmax(anc_ref[...])
    n_prob = _softmax(nei_ref[...])
    ap_ref[...] = a_prob
    augp_ref[...] = aug_scr[pl.ds(i * BLK, BLK), :]

    sim = jnp.sum(a_prob * n_prob, axis=1)
    cons_part = jnp.sum(-jnp.maximum(jnp.log(sim), -100.0))
    vec_ref[0, :] += jnp.sum(a_prob, axis=0)
    vec_ref[1, :] += jnp.sum(a_prob * a_prob, axis=0)
    vec_ref[2, :] += jnp.sum(n_prob * n_prob, axis=0)
    g_ref[...] += jnp.dot(a_prob.T, n_prob, preferred_element_type=jnp.float32)

    a = af_ref[...]
    a_nrm = jnp.sum(a * a, axis=1, keepdims=True)
    d2 = a_nrm + gn_ref[0:1, :] - 2.0 * jnp.dot(
        a, gf_ref[...].T, preferred_element_type=jnp.float32)
    d2c = jnp.maximum(d2, 0.0)

    col = jax.lax.broadcasted_iota(jnp.int32, (BLK, B), 1)
    bits = jax.lax.bitcast_convert_type(d2c, jnp.int32)
    packed = jax.lax.bitcast_convert_type(
        jnp.bitwise_or(jnp.bitwise_and(bits, jnp.int32(~0xFFF)), col),
        jnp.float32)
    fmax = jnp.float32(3.4028235e38)
    mins = []
    for k in range(K):
        m = jnp.min(packed, axis=1, keepdims=True)
        packed = jnp.where(packed == m, fmax, packed)
        mins.append(m)

    vals_p = jnp.concatenate(mins, axis=1)
    bits_k = jax.lax.bitcast_convert_type(vals_p, jnp.int32)
    v2 = jax.lax.bitcast_convert_type(
        jnp.bitwise_and(bits_k, jnp.int32(~0xFFF)), jnp.float32)
    r = jnp.sqrt(v2[:, 1:2])
    w_tiny = jnp.clip(2.0 - jnp.sqrt(v2) / r, 0.0, 1.0)
    s_ref[0] += cons_part
    s_ref[1] += jnp.sum(w_tiny)

    idx_f = jnp.bitwise_and(bits_k, jnp.int32(0xFFF)).astype(jnp.float32)
    pad = jnp.zeros((BLK, KP - K), jnp.float32)
    idx16 = jnp.concatenate([idx_f, pad], axis=1)
    w16 = jnp.concatenate([w_tiny, pad], axis=1)
    rid = jax.lax.broadcasted_iota(jnp.int32, (BLK, BLK), 0)
    cid = jax.lax.broadcasted_iota(jnp.int32, (BLK, BLK), 1)
    eye = jnp.where(rid == cid, 1.0, 0.0).astype(jnp.float32)
    idxf_ref[...] = jnp.dot(idx16.T, eye, preferred_element_type=jnp.float32)
    wf_ref[...] = jnp.dot(w16.T, eye, preferred_element_type=jnp.float32)

    @pl.when(i == NBLK - 1)
    def _fin():
        bsq = jnp.float32(B) * jnp.float32(B)
        colsum_a = vec_ref[0, :]
        mean_ip = jnp.sum(colsum_a * vec_ref[3, :]) / bsq
        glob_part = 1.0 - mean_ip - s_ref[1] / bsq

        mprob = jnp.maximum(colsum_a / jnp.float32(B), EPS)
        ent = -jnp.sum(mprob * jnp.log(mprob))

        na = jnp.maximum(jnp.sqrt(vec_ref[1, :]), 1e-12)
        np_ = jnp.maximum(jnp.sqrt(vec_ref[2, :]), 1e-12)
        sim_cc = g_ref[...] / (na[:, None] * np_[None, :])
        mx = jnp.max(sim_cc, axis=1, keepdims=True)
        lse = jnp.log(jnp.sum(jnp.exp(sim_cc - mx), axis=1, keepdims=True)) + mx
        rid2 = jax.lax.broadcasted_iota(jnp.int32, (C, C), 0)
        cid2 = jax.lax.broadcasted_iota(jnp.int32, (C, C), 1)
        diag_sum = jnp.sum(jnp.where(rid2 == cid2, sim_cc, 0.0))
        ce = (jnp.sum(lse) - diag_sum) / jnp.float32(C)

        cons = s_ref[0] / jnp.float32(B)
        tp_ref[...] = jnp.reshape(
            cons - ENTROPY_WEIGHT * ent + ce + glob_part, (1, 1))
        cons_ref[...] = jnp.reshape(cons, (1, 1))
        ent_ref[...] = jnp.reshape(ent, (1, 1))


_SC_MESH = plsc.VectorSubcoreMesh(core_axis_name="c", subcore_axis_name="s")


@functools.partial(
    pl.kernel,
    mesh=_SC_MESH,
    out_type=jax.ShapeDtypeStruct((NWORK, C), jnp.float32),
    scratch_types=[
        pltpu.VMEM((KP, RPW), jnp.float32),
        pltpu.VMEM((KP, RPW), jnp.int32),
        pltpu.VMEM((KP, RPW), jnp.float32),
        pltpu.VMEM((2 * RPW, C), jnp.float32),
        pltpu.VMEM((RPW, C), jnp.float32),
        pltpu.VMEM((C,), jnp.float32),
        pltpu.SemaphoreType.DMA,
        pltpu.SemaphoreType.DMA,
    ],
)
def _sc_gather(idxf_hbm, wf_hbm, augp_hbm, ap_hbm, out_hbm,
               idxf_v, idx_v, w_v, rows_v, a_v, acc_v, sem0, sem1):
    wid = lax.axis_index("s") * 2 + lax.axis_index("c")
    base = wid * RPW

    pltpu.sync_copy(idxf_hbm.at[pl.ds(0, KP), pl.ds(base, RPW)], idxf_v)
    pltpu.sync_copy(wf_hbm.at[pl.ds(0, KP), pl.ds(base, RPW)], w_v)
    pltpu.sync_copy(ap_hbm.at[pl.ds(base, RPW)], a_v)
    for k in range(K):
        for c in range(RPW // LANES):
            sl = pl.ds(c * LANES, LANES)
            idx_v[k, sl] = idxf_v[k, sl].astype(jnp.int32)

    sems = (sem0, sem1)

    def issue(k):
        buf = k % 2
        return pltpu.async_copy(augp_hbm.at[idx_v.at[k]],
                                rows_v.at[pl.ds(buf * RPW, RPW)], sems[buf])

    acc = (jnp.zeros((LANES,), jnp.float32),) * (C // LANES)
    pending = {0: issue(0)}
    for k in range(K):
        if k + 1 < K:
            pending[(k + 1) % 2] = issue(k + 1)
        pending[k % 2].wait()
        buf = k % 2

        def q_body(q, acc_in, _k=k, _buf=buf):
            qsl = pl.ds(q * LANES, LANES)
            wvec = w_v[_k, qsl]
            accs = list(acc_in)
            for l in range(LANES):
                r = q * LANES + l
                wl = wvec[l]
                for c in range(C // LANES):
                    sl = pl.ds(c * LANES, LANES)
                    accs[c] = accs[c] + wl * (
                        rows_v[_buf * RPW + r, sl] * a_v[r, sl])
            return tuple(accs)

        acc = lax.fori_loop(0, RPW // LANES, q_body, acc)
    for c in range(C // LANES):
        acc_v[pl.ds(c * LANES, LANES)] = acc[c]
    pltpu.sync_copy(acc_v, out_hbm.at[wid])


def _fin_kernel(tp_ref, wip_ref, out_ref):
    bsq = jnp.float32(B) * jnp.float32(B)
    out_ref[...] = tp_ref[...] + 2.0 * jnp.sum(wip_ref[...]) / bsq


@jax.jit
def kernel(anchors_features, augments_features, anchors, neighbors, augments):
    full = pl.BlockSpec((B, D), lambda i: (0, 0))
    blk = pl.BlockSpec((BLK, C), lambda i: (i, 0))
    out1 = pl.BlockSpec((1, 1), lambda i: (0, 0))
    kblk = pl.BlockSpec((KP, BLK), lambda i: (0, i))
    tp, cons, ent, idxf, wf, augp, ap = pl.pallas_call(
        _main_kernel,
        grid=(NBLK,),
        in_specs=[pl.BlockSpec((BLK, D), lambda i: (i, 0)), full, blk, blk, full],
        out_specs=[out1, out1, out1, kblk, kblk, blk, blk],
        out_shape=[
            jax.ShapeDtypeStruct((1, 1), jnp.float32),
            jax.ShapeDtypeStruct((1, 1), jnp.float32),
            jax.ShapeDtypeStruct((1, 1), jnp.float32),
            jax.ShapeDtypeStruct((KP, B), jnp.float32),
            jax.ShapeDtypeStruct((KP, B), jnp.float32),
            jax.ShapeDtypeStruct((B, C), jnp.float32),
            jax.ShapeDtypeStruct((B, C), jnp.float32),
        ],
        scratch_shapes=[
            pltpu.VMEM((8, C), jnp.float32),
            pltpu.VMEM((C, C), jnp.float32),
            pltpu.VMEM((B, C), jnp.float32),
            pltpu.VMEM((8, B), jnp.float32),
            pltpu.SMEM((4,), jnp.float32),
        ],
    )(anchors_features, augments_features, anchors, neighbors, augments)

    wip_part = _sc_gather(idxf, wf, augp, ap)

    total = pl.pallas_call(
        _fin_kernel,
        out_shape=jax.ShapeDtypeStruct((1, 1), jnp.float32),
    )(tp, wip_part)
    return (total[0, 0], cons[0, 0], ent[0, 0])
